# Initial kernel scaffold; baseline (speedup 1.0000x reference)
import jax
import jax.numpy as jnp
from jax.experimental import pallas as pl


def kernel(target, permutation):
    raise NotImplementedError("write your pallas kernel here")


# speedup vs baseline: 1.1884x; 1.1884x over previous
import jax
import jax.numpy as jnp
from jax import lax
from jax.experimental import pallas as pl
from jax.experimental.pallas import tpu as pltpu
from jax.experimental.pallas import tpu_sc as plsc

BATCH = 16384
D = 128
L = 16
NC = 2
NS = 16
NW = NC * NS
ROWS_PER_W = BATCH // NW
CHUNK = 128
NCHUNKS = ROWS_PER_W // CHUNK


def _permute_body(tgt_hbm, perm_hbm, out_hbm, perm_v, in_v, out_v):
    wid = lax.axis_index("s") * NC + lax.axis_index("c")
    pltpu.sync_copy(perm_hbm, perm_v)
    pslices = [perm_v[pl.ds(j * L, L)] for j in range(D // L)]
    row0 = wid * ROWS_PER_W

    for c in range(NCHUNKS):
        base = (row0 + c * CHUNK) * D
        pltpu.sync_copy(tgt_hbm.at[pl.ds(base, CHUNK * D)], in_v)

        def body(r, carry):
            roff = r * D
            for j in range(D // L):
                idx = pslices[j] + roff
                vals = plsc.load_gather(in_v, [idx])
                out_v[pl.ds(roff + j * L, L)] = vals
            return carry

        lax.fori_loop(0, CHUNK, body, 0)
        pltpu.sync_copy(out_v, out_hbm.at[pl.ds(base, CHUNK * D)])


def kernel(target, permutation):
    mesh = plsc.VectorSubcoreMesh(core_axis_name="c", subcore_axis_name="s")
    k = pl.kernel(
        _permute_body,
        out_type=jax.ShapeDtypeStruct((BATCH * D,), jnp.float32),
        mesh=mesh,
        compiler_params=pltpu.CompilerParams(needs_layout_passes=False),
        scratch_types=[
            pltpu.VMEM((D,), jnp.int32),
            pltpu.VMEM((CHUNK * D,), jnp.float32),
            pltpu.VMEM((CHUNK * D,), jnp.float32),
        ],
    )
    out_flat = k(target.reshape(-1), permutation)
    return out_flat.reshape(BATCH, D)


# speedup vs baseline: 1.8840x; 1.5854x over previous
import jax
import jax.numpy as jnp
from jax import lax
from jax.experimental import pallas as pl
from jax.experimental.pallas import tpu as pltpu
from jax.experimental.pallas import tpu_sc as plsc

BATCH = 16384
D = 128
L = 16
NC = 2
NS = 16
NW = NC * NS
ROWS_PER_W = BATCH // NW
CHUNK = 128
NCHUNKS = ROWS_PER_W // CHUNK


def _permute_body(tgt_hbm, perm_hbm, out_hbm, perm_v,
                  in0, in1, out0, out1,
                  sem_in0, sem_in1, sem_out0, sem_out1):
    wid = lax.axis_index("s") * NC + lax.axis_index("c")
    pltpu.sync_copy(perm_hbm, perm_v)
    pslices = [perm_v[pl.ds(j * L, L)] for j in range(D // L)]
    row0 = wid * ROWS_PER_W

    in_bufs = (in0, in1)
    out_bufs = (out0, out1)
    sem_in = (sem_in0, sem_in1)
    sem_out = (sem_out0, sem_out1)

    def hbm_slice(c):
        return pl.ds((row0 + c * CHUNK) * D, CHUNK * D)

    def compute(in_ref, out_ref):
        @plsc.parallel_loop(0, CHUNK, unroll=4)
        def _(r):
            roff = r * D
            for j in range(D // L):
                idx = pslices[j] + roff
                out_ref[pl.ds(roff + j * L, L)] = plsc.load_gather(in_ref, [idx])

    in_dma = [None] * NCHUNKS
    out_dma = [None] * NCHUNKS
    in_dma[0] = pltpu.async_copy(tgt_hbm.at[hbm_slice(0)], in_bufs[0], sem_in[0])
    for c in range(NCHUNKS):
        b = c % 2
        if c + 1 < NCHUNKS:
            in_dma[c + 1] = pltpu.async_copy(
                tgt_hbm.at[hbm_slice(c + 1)], in_bufs[1 - b], sem_in[1 - b])
        in_dma[c].wait()
        if c >= 2:
            out_dma[c - 2].wait()
        compute(in_bufs[b], out_bufs[b])
        out_dma[c] = pltpu.async_copy(out_bufs[b], out_hbm.at[hbm_slice(c)],
                                      sem_out[b])
    for c in range(max(0, NCHUNKS - 2), NCHUNKS):
        out_dma[c].wait()


def kernel(target, permutation):
    mesh = plsc.VectorSubcoreMesh(core_axis_name="c", subcore_axis_name="s")
    k = pl.kernel(
        _permute_body,
        out_type=jax.ShapeDtypeStruct((BATCH * D,), jnp.float32),
        mesh=mesh,
        compiler_params=pltpu.CompilerParams(needs_layout_passes=False),
        scratch_types=[
            pltpu.VMEM((D,), jnp.int32),
            pltpu.VMEM((CHUNK * D,), jnp.float32),
            pltpu.VMEM((CHUNK * D,), jnp.float32),
            pltpu.VMEM((CHUNK * D,), jnp.float32),
            pltpu.VMEM((CHUNK * D,), jnp.float32),
            pltpu.SemaphoreType.DMA,
            pltpu.SemaphoreType.DMA,
            pltpu.SemaphoreType.DMA,
            pltpu.SemaphoreType.DMA,
        ],
    )
    out_flat = k(target.reshape(-1), permutation)
    return out_flat.reshape(BATCH, D)


# speedup vs baseline: 1.8980x; 1.0074x over previous
import jax
import jax.numpy as jnp
from jax import lax
from jax.experimental import pallas as pl
from jax.experimental.pallas import tpu as pltpu
from jax.experimental.pallas import tpu_sc as plsc

BATCH = 16384
D = 128
L = 16
NC = 2
NS = 16
NW = NC * NS
ROWS_PER_W = BATCH // NW
CHUNK = 128
NCHUNKS = ROWS_PER_W // CHUNK


def _permute_body(tgt_hbm, perm_hbm, out_hbm, perm_v,
                  in0, in1, out0, out1,
                  sem_in0, sem_in1, sem_out0, sem_out1):
    wid = lax.axis_index("s") * NC + lax.axis_index("c")
    pltpu.sync_copy(perm_hbm, perm_v)
    pslices = [perm_v[pl.ds(j * L, L)] for j in range(D // L)]
    row0 = wid * ROWS_PER_W

    in_bufs = (in0, in1)
    out_bufs = (out0, out1)
    sem_in = (sem_in0, sem_in1)
    sem_out = (sem_out0, sem_out1)

    def rows(c):
        return pl.ds(row0 + c * CHUNK, CHUNK)

    def compute(in_ref, out_ref):
        @plsc.parallel_loop(0, CHUNK, unroll=4)
        def _(r):
            rvec = jnp.full((L,), 0, jnp.int32) + r
            for j in range(D // L):
                out_ref[r, pl.ds(j * L, L)] = plsc.load_gather(
                    in_ref, [rvec, pslices[j]])

    in_dma = [None] * NCHUNKS
    out_dma = [None] * NCHUNKS
    in_dma[0] = pltpu.async_copy(tgt_hbm.at[rows(0)], in_bufs[0], sem_in[0])
    for c in range(NCHUNKS):
        b = c % 2
        if c + 1 < NCHUNKS:
            in_dma[c + 1] = pltpu.async_copy(
                tgt_hbm.at[rows(c + 1)], in_bufs[1 - b], sem_in[1 - b])
        in_dma[c].wait()
        if c >= 2:
            out_dma[c - 2].wait()
        compute(in_bufs[b], out_bufs[b])
        out_dma[c] = pltpu.async_copy(out_bufs[b], out_hbm.at[rows(c)],
                                      sem_out[b])
    for c in range(max(0, NCHUNKS - 2), NCHUNKS):
        out_dma[c].wait()


def kernel(target, permutation):
    mesh = plsc.VectorSubcoreMesh(core_axis_name="c", subcore_axis_name="s")
    k = pl.kernel(
        _permute_body,
        out_type=jax.ShapeDtypeStruct((BATCH, D), jnp.float32),
        mesh=mesh,
        compiler_params=pltpu.CompilerParams(needs_layout_passes=False),
        scratch_types=[
            pltpu.VMEM((D,), jnp.int32),
            pltpu.VMEM((CHUNK, D), jnp.float32),
            pltpu.VMEM((CHUNK, D), jnp.float32),
            pltpu.VMEM((CHUNK, D), jnp.float32),
            pltpu.VMEM((CHUNK, D), jnp.float32),
            pltpu.SemaphoreType.DMA,
            pltpu.SemaphoreType.DMA,
            pltpu.SemaphoreType.DMA,
            pltpu.SemaphoreType.DMA,
        ],
    )
    return k(target, permutation)
